# baseline (speedup 1.0000x reference)
import jax
import jax.numpy as jnp
from jax import lax
from jax.experimental import pallas as pl
from jax.experimental.pallas import tpu as pltpu
from jax.experimental.pallas import tpu_sc as plsc

DIM = 64
HIST = 200
PAD = 208
NCHUNK = PAD // 16
GROWS = 224
UROW = PAD


def _gather_body(uidx_s, idx_s, ut_hbm, it_hbm, gout, blocks_vm, rows_vm,
                 sem, usem, osem):
    ublk = pl.multiple_of((uidx_s[0] // 8) * 8, 8)
    pltpu.async_copy(
        ut_hbm.at[pl.ds(ublk, 8), :],
        blocks_vm.at[pl.ds(UROW * 8, 8), :], usem)

    def issue(l, carry):
        blk = pl.multiple_of((idx_s[l] // 8) * 8, 8)
        pltpu.async_copy(
            it_hbm.at[pl.ds(blk, 8), :],
            blocks_vm.at[pl.ds(l * 8, 8), :], sem)
        return carry

    lax.fori_loop(0, PAD, issue, 0)

    def drain(l, carry):
        pltpu.make_async_copy(
            it_hbm.at[pl.ds(0, 8), :],
            blocks_vm.at[pl.ds(0, 8), :], sem).wait()
        return carry

    lax.fori_loop(0, PAD, drain, 0)
    pltpu.make_async_copy(
        ut_hbm.at[pl.ds(0, 8), :],
        blocks_vm.at[pl.ds(0, 8), :], usem).wait()

    def compact(l, carry):
        rows_vm[l, :] = blocks_vm[l * 8 + (idx_s[l] % 8), :]
        return carry

    lax.fori_loop(0, PAD, compact, 0)
    rows_vm[UROW, :] = blocks_vm[UROW * 8 + (uidx_s[0] % 8), :]
    pltpu.async_copy(rows_vm, gout, osem).wait()


def _attn_body(g_hbm, wp_hbm, out_hbm, rows_v, wp_v, a_v):
    cid = lax.axis_index("c")
    sid = lax.axis_index("s")
    is_main = jnp.logical_and(cid == 0, sid == 0)

    @pl.when(is_main)
    def _():
        pltpu.sync_copy(g_hbm, rows_v)
        pltpu.sync_copy(wp_hbm, wp_v)

        def fold_step(kb, gs):
            w2c = wp_v[DIM, pl.ds(DIM + kb * 16, 16)]
            for i in range(16):
                k = kb * 16 + i
                w2k = w2c[i]
                gs = tuple(
                    gs[c] + wp_v[k, pl.ds(c * 16, 16)] * w2k
                    for c in range(8))
            return gs

        zeros = jnp.zeros((16,), jnp.float32)
        g = lax.fori_loop(0, 4, fold_step, (zeros,) * 8)

        lane = lax.iota(jnp.int32, 16)

        def _shuf(v, sh):
            return v.at[lane ^ sh].get(mode="promise_in_bounds")

        def hsum(v):
            for sh in (8, 4, 2, 1):
                v = v + _shuf(v, sh)
            return v

        def hmax(v):
            for sh in (8, 4, 2, 1):
                v = jnp.maximum(v, _shuf(v, sh))
            return v

        uacc = (rows_v[UROW, pl.ds(0, 16)] * g[0]
                + rows_v[UROW, pl.ds(16, 16)] * g[1]
                + rows_v[UROW, pl.ds(32, 16)] * g[2]
                + rows_v[UROW, pl.ds(48, 16)] * g[3])
        bacc = (wp_v[DIM, pl.ds(0, 16)] * wp_v[DIM, pl.ds(DIM, 16)]
                + wp_v[DIM, pl.ds(16, 16)] * wp_v[DIM, pl.ds(DIM + 16, 16)]
                + wp_v[DIM, pl.ds(32, 16)] * wp_v[DIM, pl.ds(DIM + 32, 16)]
                + wp_v[DIM, pl.ds(48, 16)] * wp_v[DIM, pl.ds(DIM + 48, 16)])
        b2c = wp_v[DIM + 1, pl.ds(0, 16)]
        const = hsum(uacc + bacc + b2c)

        g4, g5, g6, g7 = g[4], g[5], g[6], g[7]
        lane_is = [lane == i for i in range(16)]

        def chunk_step(c, carry):
            base = c * 16
            av = jnp.zeros((16,), jnp.float32)
            for i in range(16):
                r = (rows_v[base + i, pl.ds(0, 16)] * g4
                     + rows_v[base + i, pl.ds(16, 16)] * g5
                     + rows_v[base + i, pl.ds(32, 16)] * g6
                     + rows_v[base + i, pl.ds(48, 16)] * g7)
                av = jnp.where(lane_is[i], hsum(r), av)
            s = av + const
            s = jnp.where(s >= 0.0, s, 0.01 * s)
            a_v[pl.ds(base, 16)] = s
            return carry

        lax.fori_loop(0, NCHUNK, chunk_step, 0)

        tail_mask = lane < (HIST - (NCHUNK - 1) * 16)
        neg_big = jnp.full((16,), -jnp.inf, jnp.float32)

        mvec = neg_big
        for c in range(NCHUNK):
            chunk = a_v[pl.ds(c * 16, 16)]
            if c == NCHUNK - 1:
                chunk = jnp.where(tail_mask, chunk, neg_big)
            mvec = jnp.maximum(mvec, chunk)
        m = hmax(mvec)

        svec = jnp.zeros((16,), jnp.float32)
        for c in range(NCHUNK):
            chunk = a_v[pl.ds(c * 16, 16)]
            e = jnp.exp(chunk - m)
            if c == NCHUNK - 1:
                e = jnp.where(tail_mask, e, 0.0)
            a_v[pl.ds(c * 16, 16)] = e
            svec = svec + e
        inv = 1.0 / hsum(svec)

        for c in range(NCHUNK):
            a_v[pl.ds(c * 16, 16)] = a_v[pl.ds(c * 16, 16)] * inv

        pltpu.sync_copy(a_v.at[pl.ds(0, HIST)], out_hbm)


@jax.jit
def _attention(uidx16, idx_all, user_table, item_table, wpack):
    gathered = pl.pallas_call(
        _gather_body,
        out_shape=jax.ShapeDtypeStruct((GROWS, DIM), jnp.float32),
        in_specs=[
            pl.BlockSpec(memory_space=pltpu.SMEM),
            pl.BlockSpec(memory_space=pltpu.SMEM),
            pl.BlockSpec(memory_space=pl.ANY),
            pl.BlockSpec(memory_space=pl.ANY),
        ],
        out_specs=pl.BlockSpec(memory_space=pl.ANY),
        scratch_shapes=[
            pltpu.VMEM((GROWS * 8, DIM), jnp.float32),
            pltpu.VMEM((GROWS, DIM), jnp.float32),
            pltpu.SemaphoreType.DMA,
            pltpu.SemaphoreType.DMA,
            pltpu.SemaphoreType.DMA,
        ],
    )(uidx16, idx_all, user_table, item_table)

    run = pl.kernel(
        _attn_body,
        mesh=plsc.VectorSubcoreMesh(core_axis_name="c", subcore_axis_name="s"),
        out_type=jax.ShapeDtypeStruct((HIST,), jnp.float32),
        compiler_params=pltpu.CompilerParams(use_tc_tiling_on_sc=True),
        scratch_types=[
            pltpu.VMEM((GROWS, DIM), jnp.float32),
            pltpu.VMEM((72, 2 * DIM), jnp.float32),
            pltpu.VMEM((PAD,), jnp.float32),
        ],
    )
    return run(gathered, wpack)


def kernel(user_indice, interacted_item_indices, user_table, item_table,
           W1, b1, W2, b2):
    idx_all = jnp.concatenate(
        [interacted_item_indices.astype(jnp.int32),
         jnp.zeros((PAD - HIST,), jnp.int32)])
    uidx16 = jnp.full((16,), user_indice, dtype=jnp.int32)
    row_bw = jnp.concatenate([b1, W2.reshape(DIM)])[None, :]
    row_b2 = jnp.pad(b2, (0, 2 * DIM - 1))[None, :]
    wpack = jnp.concatenate(
        [W1.T, row_bw, row_b2,
         jnp.zeros((72 - DIM - 2, 2 * DIM), jnp.float32)], axis=0)
    return _attention(uidx16, idx_all, user_table, item_table, wpack)


# speedup vs baseline: 15.5326x; 15.5326x over previous
import jax
import jax.numpy as jnp
from jax import lax
from jax.experimental import pallas as pl
from jax.experimental.pallas import tpu as pltpu
from jax.experimental.pallas import tpu_sc as plsc

DIM = 64
HIST = 200
PAD = 208
NCHUNK = PAD // 16
NROWS = 1000000
TAILSTART = (NROWS // 128) * 128
AOUT = 224


def _score_body(uidx_s, idx_s, idx2d_ref, utT, itT, w1_ref, w2_ref,
                utail_ref, itail_ref, araw_ref, panels, upanel, sem, usem):
    def issue(l, carry):
        i = idx_s[l]
        s = jnp.where(i >= TAILSTART, 0, (i // 128) * 128)
        s = pl.multiple_of(s, 128)
        pltpu.async_copy(
            itT.at[:, pl.ds(s, 128)], panels.at[l], sem)
        return carry

    lax.fori_loop(0, PAD, issue, 0)

    ui = uidx_s[0]
    us = jnp.where(ui >= TAILSTART, 0, (ui // 128) * 128)
    us = pl.multiple_of(us, 128)
    pltpu.async_copy(utT.at[:, pl.ds(us, 128)], upanel, usem)

    gvec = jnp.sum(w1_ref[...] * w2_ref[...][:, 0][None, :], axis=1)

    def drain(l, carry):
        pltpu.make_async_copy(
            itT.at[:, pl.ds(0, 128)], panels.at[0], sem).wait()
        return carry

    lax.fori_loop(0, PAD, drain, 0)
    pltpu.make_async_copy(utT.at[:, pl.ds(0, 128)], upanel, usem).wait()

    acc = jnp.zeros((PAD, 128), jnp.float32)
    htail = jnp.zeros((1, 128), jnp.float32)
    uacc = jnp.zeros((1, 128), jnp.float32)
    utailacc = jnp.zeros((1, 128), jnp.float32)
    for d in range(DIM):
        gi = gvec[DIM + d]
        gu = gvec[d]
        acc = acc + panels[:, d, :] * gi
        htail = htail + itail_ref[d, :][None, :] * gi
        uacc = uacc + upanel[d, :][None, :] * gu
        utailacc = utailacc + utail_ref[d, :][None, :] * gu

    idx2d = idx2d_ref[...]
    tmask = idx2d >= TAILSTART
    lanes = jnp.where(tmask, idx2d - TAILSTART, idx2d & 127)
    hfin = jnp.where(tmask, htail, acc)
    lmask = lax.broadcasted_iota(jnp.int32, (PAD, 128), 1) == lanes
    araw_ref[pl.ds(0, PAD)] = jnp.sum(
        jnp.where(lmask, hfin, 0.0), axis=1)

    ut_flag = ui >= TAILSTART
    uh = jnp.where(ut_flag, utailacc, uacc)
    ul = jnp.where(ut_flag, ui - TAILSTART, ui & 127)
    ulmask = lax.broadcasted_iota(jnp.int32, (1, 128), 1) == ul
    araw_ref[pl.ds(PAD, 1)] = jnp.sum(jnp.where(ulmask, uh, 0.0), axis=1)
    araw_ref[pl.ds(PAD + 1, AOUT - PAD - 1)] = jnp.zeros(
        (AOUT - PAD - 1,), jnp.float32)


def _softmax_body(araw_hbm, bpack_hbm, out_hbm, ar_v, bp_v):
    cid = lax.axis_index("c")
    sid = lax.axis_index("s")
    is_main = jnp.logical_and(cid == 0, sid == 0)

    @pl.when(is_main)
    def _():
        pltpu.sync_copy(araw_hbm, ar_v)
        pltpu.sync_copy(bpack_hbm, bp_v)

        lane = lax.iota(jnp.int32, 16)

        def _shuf(v, sh):
            return v.at[lane ^ sh].get(mode="promise_in_bounds")

        def hsum(v):
            for sh in (8, 4, 2, 1):
                v = v + _shuf(v, sh)
            return v

        def hmax(v):
            for sh in (8, 4, 2, 1):
                v = jnp.maximum(v, _shuf(v, sh))
            return v

        bacc = (bp_v[0, pl.ds(0, 16)] * bp_v[0, pl.ds(DIM, 16)]
                + bp_v[0, pl.ds(16, 16)] * bp_v[0, pl.ds(DIM + 16, 16)]
                + bp_v[0, pl.ds(32, 16)] * bp_v[0, pl.ds(DIM + 32, 16)]
                + bp_v[0, pl.ds(48, 16)] * bp_v[0, pl.ds(DIM + 48, 16)])
        b2c = bp_v[1, pl.ds(0, 16)]
        uch = ar_v[pl.ds(PAD, 16)]
        usplat = uch.at[jnp.zeros((16,), jnp.int32)].get(
            mode="promise_in_bounds")
        const = usplat + hsum(bacc + b2c)

        tail_mask = lane < (HIST - (NCHUNK - 1) * 16)
        neg_big = jnp.full((16,), -jnp.inf, jnp.float32)

        mvec = neg_big
        for c in range(NCHUNK):
            s = ar_v[pl.ds(c * 16, 16)] + const
            s = jnp.where(s >= 0.0, s, 0.01 * s)
            ar_v[pl.ds(c * 16, 16)] = s
            if c == NCHUNK - 1:
                s = jnp.where(tail_mask, s, neg_big)
            mvec = jnp.maximum(mvec, s)
        m = hmax(mvec)

        svec = jnp.zeros((16,), jnp.float32)
        for c in range(NCHUNK):
            chunk = ar_v[pl.ds(c * 16, 16)]
            e = jnp.exp(chunk - m)
            if c == NCHUNK - 1:
                e = jnp.where(tail_mask, e, 0.0)
            ar_v[pl.ds(c * 16, 16)] = e
            svec = svec + e
        inv = 1.0 / hsum(svec)

        for c in range(NCHUNK):
            ar_v[pl.ds(c * 16, 16)] = ar_v[pl.ds(c * 16, 16)] * inv

        pltpu.sync_copy(ar_v.at[pl.ds(0, HIST)], out_hbm)


@jax.jit
def _attention(uidx16, idx_all, idx2d, utT, itT, W1, W2, ut_tail, it_tail,
               bpack):
    araw = pl.pallas_call(
        _score_body,
        out_shape=jax.ShapeDtypeStruct((AOUT,), jnp.float32),
        in_specs=[
            pl.BlockSpec(memory_space=pltpu.SMEM),
            pl.BlockSpec(memory_space=pltpu.SMEM),
            pl.BlockSpec(memory_space=pltpu.VMEM),
            pl.BlockSpec(memory_space=pl.ANY),
            pl.BlockSpec(memory_space=pl.ANY),
            pl.BlockSpec(memory_space=pltpu.VMEM),
            pl.BlockSpec(memory_space=pltpu.VMEM),
            pl.BlockSpec(memory_space=pltpu.VMEM),
            pl.BlockSpec(memory_space=pltpu.VMEM),
        ],
        out_specs=pl.BlockSpec(memory_space=pltpu.VMEM),
        scratch_shapes=[
            pltpu.VMEM((PAD, DIM, 128), jnp.float32),
            pltpu.VMEM((DIM, 128), jnp.float32),
            pltpu.SemaphoreType.DMA,
            pltpu.SemaphoreType.DMA,
        ],
    )(uidx16, idx_all, idx2d, utT, itT, W1, W2, ut_tail, it_tail)

    run = pl.kernel(
        _softmax_body,
        mesh=plsc.VectorSubcoreMesh(core_axis_name="c", subcore_axis_name="s"),
        out_type=jax.ShapeDtypeStruct((HIST,), jnp.float32),
        compiler_params=pltpu.CompilerParams(use_tc_tiling_on_sc=True),
        scratch_types=[
            pltpu.VMEM((AOUT,), jnp.float32),
            pltpu.VMEM((2, 2 * DIM), jnp.float32),
        ],
    )
    return run(araw, bpack)


def kernel(user_indice, interacted_item_indices, user_table, item_table,
           W1, b1, W2, b2):
    idx_all = jnp.concatenate(
        [interacted_item_indices.astype(jnp.int32),
         jnp.zeros((PAD - HIST,), jnp.int32)])
    idx2d = idx_all.reshape(PAD, 1)
    uidx16 = jnp.full((16,), user_indice, dtype=jnp.int32)
    utT = user_table.T
    itT = item_table.T
    ntail = NROWS - TAILSTART
    ut_tail = jnp.pad(utT[:, TAILSTART:], ((0, 0), (0, 128 - ntail)))
    it_tail = jnp.pad(itT[:, TAILSTART:], ((0, 0), (0, 128 - ntail)))
    row_bw = jnp.concatenate([b1, W2.reshape(DIM)])
    row_b2 = jnp.pad(b2, (0, 2 * DIM - 1))
    bpack = jnp.stack([row_bw, row_b2])
    return _attention(uidx16, idx_all, idx2d, utT, itT, W1, W2, ut_tail,
                      it_tail, bpack)


# speedup vs baseline: 16.1579x; 1.0403x over previous
import jax
import jax.numpy as jnp
from jax import lax
from jax.experimental import pallas as pl
from jax.experimental.pallas import tpu as pltpu
from jax.experimental.pallas import tpu_sc as plsc

DIM = 64
HIST = 200
PAD = 208
NCHUNK = PAD // 16
NROWS = 1000000
TAILSTART = (NROWS // 128) * 128
AOUT = 224


def _score_body(uidx_s, idx_s, idx2d_ref, utT, itT, w1_ref, w2_ref,
                b1_ref, b2_ref, utail_ref, itail_ref, araw_ref, panels,
                upanel, sem, usem):
    def issue(l, carry):
        i = idx_s[l]
        s = jnp.where(i >= TAILSTART, 0, (i // 128) * 128)
        s = pl.multiple_of(s, 128)
        pltpu.async_copy(
            itT.at[:, pl.ds(s, 128)], panels.at[l], sem)
        return carry

    lax.fori_loop(0, PAD, issue, 0)

    ui = uidx_s[0]
    us = jnp.where(ui >= TAILSTART, 0, (ui // 128) * 128)
    us = pl.multiple_of(us, 128)
    pltpu.async_copy(utT.at[:, pl.ds(us, 128)], upanel, usem)

    gvec = jnp.sum(w1_ref[...] * w2_ref[...][:, 0][None, :], axis=1)

    def drain(l, carry):
        pltpu.make_async_copy(
            itT.at[:, pl.ds(0, 128)], panels.at[0], sem).wait()
        return carry

    lax.fori_loop(0, PAD, drain, 0)
    pltpu.make_async_copy(utT.at[:, pl.ds(0, 128)], upanel, usem).wait()

    acc = jnp.zeros((PAD, 128), jnp.float32)
    htail = jnp.zeros((1, 128), jnp.float32)
    uacc = jnp.zeros((1, 128), jnp.float32)
    utailacc = jnp.zeros((1, 128), jnp.float32)
    for d in range(DIM):
        gi = gvec[DIM + d]
        gu = gvec[d]
        acc = acc + panels[:, d, :] * gi
        htail = htail + itail_ref[d, :][None, :] * gi
        uacc = uacc + upanel[d, :][None, :] * gu
        utailacc = utailacc + utail_ref[d, :][None, :] * gu

    ut_flag = ui >= TAILSTART
    uh = jnp.where(ut_flag, utailacc, uacc)
    ul = jnp.where(ut_flag, ui - TAILSTART, ui & 127)
    ulmask = lax.broadcasted_iota(jnp.int32, (1, 128), 1) == ul
    uval = jnp.sum(jnp.where(ulmask, uh, 0.0))

    const = uval + jnp.sum(b1_ref[...] * w2_ref[...][:, 0]) + b2_ref[0]

    idx2d = idx2d_ref[...]
    tmask = idx2d >= TAILSTART
    lanes = jnp.where(tmask, idx2d - TAILSTART, idx2d & 127)
    hfin = jnp.where(tmask, htail, acc)
    lmask = lax.broadcasted_iota(jnp.int32, (PAD, 128), 1) == lanes
    araw_ref[pl.ds(0, PAD)] = jnp.sum(
        jnp.where(lmask, hfin, 0.0), axis=1) + const
    araw_ref[pl.ds(PAD, AOUT - PAD)] = jnp.zeros(
        (AOUT - PAD,), jnp.float32)


def _softmax_body(araw_hbm, out_hbm, ar_v):
    cid = lax.axis_index("c")
    sid = lax.axis_index("s")
    is_main = jnp.logical_and(cid == 0, sid == 0)

    @pl.when(is_main)
    def _():
        pltpu.sync_copy(araw_hbm, ar_v)

        lane = lax.iota(jnp.int32, 16)

        def _shuf(v, sh):
            return v.at[lane ^ sh].get(mode="promise_in_bounds")

        def hsum(v):
            for sh in (8, 4, 2, 1):
                v = v + _shuf(v, sh)
            return v

        def hmax(v):
            for sh in (8, 4, 2, 1):
                v = jnp.maximum(v, _shuf(v, sh))
            return v

        tail_mask = lane < (HIST - (NCHUNK - 1) * 16)
        neg_big = jnp.full((16,), -jnp.inf, jnp.float32)

        mvec = neg_big
        for c in range(NCHUNK):
            s = ar_v[pl.ds(c * 16, 16)]
            s = jnp.where(s >= 0.0, s, 0.01 * s)
            ar_v[pl.ds(c * 16, 16)] = s
            if c == NCHUNK - 1:
                s = jnp.where(tail_mask, s, neg_big)
            mvec = jnp.maximum(mvec, s)
        m = hmax(mvec)

        svec = jnp.zeros((16,), jnp.float32)
        for c in range(NCHUNK):
            chunk = ar_v[pl.ds(c * 16, 16)]
            e = jnp.exp(chunk - m)
            if c == NCHUNK - 1:
                e = jnp.where(tail_mask, e, 0.0)
            ar_v[pl.ds(c * 16, 16)] = e
            svec = svec + e
        inv = 1.0 / hsum(svec)

        for c in range(NCHUNK):
            ar_v[pl.ds(c * 16, 16)] = ar_v[pl.ds(c * 16, 16)] * inv

        pltpu.sync_copy(ar_v.at[pl.ds(0, HIST)], out_hbm)


@jax.jit
def _attention(uidx16, idx_all, idx2d, utT, itT, W1, W2, b1, b2, ut_tail,
               it_tail):
    araw = pl.pallas_call(
        _score_body,
        out_shape=jax.ShapeDtypeStruct((AOUT,), jnp.float32),
        in_specs=[
            pl.BlockSpec(memory_space=pltpu.SMEM),
            pl.BlockSpec(memory_space=pltpu.SMEM),
            pl.BlockSpec(memory_space=pltpu.VMEM),
            pl.BlockSpec(memory_space=pl.ANY),
            pl.BlockSpec(memory_space=pl.ANY),
            pl.BlockSpec(memory_space=pltpu.VMEM),
            pl.BlockSpec(memory_space=pltpu.VMEM),
            pl.BlockSpec(memory_space=pltpu.VMEM),
            pl.BlockSpec(memory_space=pltpu.SMEM),
            pl.BlockSpec(memory_space=pltpu.VMEM),
            pl.BlockSpec(memory_space=pltpu.VMEM),
        ],
        out_specs=pl.BlockSpec(memory_space=pltpu.VMEM),
        scratch_shapes=[
            pltpu.VMEM((PAD, DIM, 128), jnp.float32),
            pltpu.VMEM((DIM, 128), jnp.float32),
            pltpu.SemaphoreType.DMA,
            pltpu.SemaphoreType.DMA,
        ],
    )(uidx16, idx_all, idx2d, utT, itT, W1, W2, b1, b2, ut_tail, it_tail)

    run = pl.kernel(
        _softmax_body,
        mesh=plsc.VectorSubcoreMesh(core_axis_name="c", subcore_axis_name="s"),
        out_type=jax.ShapeDtypeStruct((HIST,), jnp.float32),
        compiler_params=pltpu.CompilerParams(use_tc_tiling_on_sc=True),
        scratch_types=[
            pltpu.VMEM((AOUT,), jnp.float32),
        ],
    )
    return run(araw)


def kernel(user_indice, interacted_item_indices, user_table, item_table,
           W1, b1, W2, b2):
    idx_all = jnp.concatenate(
        [interacted_item_indices.astype(jnp.int32),
         jnp.zeros((PAD - HIST,), jnp.int32)])
    idx2d = idx_all.reshape(PAD, 1)
    uidx16 = jnp.full((16,), user_indice, dtype=jnp.int32)
    utT = user_table.T
    itT = item_table.T
    ntail = NROWS - TAILSTART
    ut_tail = jnp.pad(utT[:, TAILSTART:], ((0, 0), (0, 128 - ntail)))
    it_tail = jnp.pad(itT[:, TAILSTART:], ((0, 0), (0, 128 - ntail)))
    return _attention(uidx16, idx_all, idx2d, utT, itT, W1, W2, b1, b2,
                      ut_tail, it_tail)


# speedup vs baseline: 21.2566x; 1.3156x over previous
import jax
import jax.numpy as jnp
from jax import lax
from jax.experimental import pallas as pl
from jax.experimental.pallas import tpu as pltpu
from jax.experimental.pallas import tpu_sc as plsc

DIM = 64
HIST = 200
PAD = 208
NCHUNK = PAD // 16
NROWS = 1000000
TAILSTART = (NROWS // 128) * 128
AOUT = 224


def _score_body(uidx_s, idx_s, idx2d_ref, utT, itT, w1_ref, w2_ref,
                b1_ref, b2_ref, utail_ref, itail_ref, araw_ref, panels,
                upanel, sem, usem):
    def issue(l, carry):
        i = idx_s[l]
        s = jnp.where(i >= TAILSTART, 0, (i // 128) * 128)
        s = pl.multiple_of(s, 128)
        pltpu.async_copy(
            itT.at[:, pl.ds(s, 128)], panels.at[:, l, :], sem)
        return carry

    lax.fori_loop(0, PAD, issue, 0)

    ui = uidx_s[0]
    us = jnp.where(ui >= TAILSTART, 0, (ui // 128) * 128)
    us = pl.multiple_of(us, 128)
    pltpu.async_copy(utT.at[:, pl.ds(us, 128)], upanel, usem)

    gvec = jnp.sum(w1_ref[...] * w2_ref[...][:, 0][None, :], axis=1)

    def drain(l, carry):
        pltpu.make_async_copy(
            itT.at[:, pl.ds(0, 128)], panels.at[:, 0, :], sem).wait()
        return carry

    lax.fori_loop(0, PAD, drain, 0)
    pltpu.make_async_copy(utT.at[:, pl.ds(0, 128)], upanel, usem).wait()

    acc = jnp.zeros((PAD, 128), jnp.float32)
    htail = jnp.zeros((1, 128), jnp.float32)
    uacc = jnp.zeros((1, 128), jnp.float32)
    utailacc = jnp.zeros((1, 128), jnp.float32)
    for d in range(DIM):
        gi = gvec[DIM + d]
        gu = gvec[d]
        acc = acc + panels[d] * gi
        htail = htail + itail_ref[d, :][None, :] * gi
        uacc = uacc + upanel[d, :][None, :] * gu
        utailacc = utailacc + utail_ref[d, :][None, :] * gu

    ut_flag = ui >= TAILSTART
    uh = jnp.where(ut_flag, utailacc, uacc)
    ul = jnp.where(ut_flag, ui - TAILSTART, ui & 127)
    ulmask = lax.broadcasted_iota(jnp.int32, (1, 128), 1) == ul
    uval = jnp.sum(jnp.where(ulmask, uh, 0.0))

    const = uval + jnp.sum(b1_ref[...] * w2_ref[...][:, 0]) + b2_ref[0]

    idx2d = idx2d_ref[...]
    tmask = idx2d >= TAILSTART
    lanes = jnp.where(tmask, idx2d - TAILSTART, idx2d & 127)
    hfin = jnp.where(tmask, htail, acc)
    lmask = lax.broadcasted_iota(jnp.int32, (PAD, 128), 1) == lanes
    araw_ref[pl.ds(0, PAD)] = jnp.sum(
        jnp.where(lmask, hfin, 0.0), axis=1) + const
    araw_ref[pl.ds(PAD, AOUT - PAD)] = jnp.zeros(
        (AOUT - PAD,), jnp.float32)


def _softmax_body(araw_hbm, out_hbm, ar_v):
    cid = lax.axis_index("c")
    sid = lax.axis_index("s")
    is_main = jnp.logical_and(cid == 0, sid == 0)

    @pl.when(is_main)
    def _():
        pltpu.sync_copy(araw_hbm, ar_v)

        lane = lax.iota(jnp.int32, 16)

        def _shuf(v, sh):
            return v.at[lane ^ sh].get(mode="promise_in_bounds")

        def hsum(v):
            for sh in (8, 4, 2, 1):
                v = v + _shuf(v, sh)
            return v

        def hmax(v):
            for sh in (8, 4, 2, 1):
                v = jnp.maximum(v, _shuf(v, sh))
            return v

        tail_mask = lane < (HIST - (NCHUNK - 1) * 16)
        neg_big = jnp.full((16,), -jnp.inf, jnp.float32)

        mvec = neg_big
        for c in range(NCHUNK):
            s = ar_v[pl.ds(c * 16, 16)]
            s = jnp.where(s >= 0.0, s, 0.01 * s)
            ar_v[pl.ds(c * 16, 16)] = s
            if c == NCHUNK - 1:
                s = jnp.where(tail_mask, s, neg_big)
            mvec = jnp.maximum(mvec, s)
        m = hmax(mvec)

        svec = jnp.zeros((16,), jnp.float32)
        for c in range(NCHUNK):
            chunk = ar_v[pl.ds(c * 16, 16)]
            e = jnp.exp(chunk - m)
            if c == NCHUNK - 1:
                e = jnp.where(tail_mask, e, 0.0)
            ar_v[pl.ds(c * 16, 16)] = e
            svec = svec + e
        inv = 1.0 / hsum(svec)

        for c in range(NCHUNK):
            ar_v[pl.ds(c * 16, 16)] = ar_v[pl.ds(c * 16, 16)] * inv

        pltpu.sync_copy(ar_v.at[pl.ds(0, HIST)], out_hbm)


@jax.jit
def _attention(uidx16, idx_all, idx2d, utT, itT, W1, W2, b1, b2, ut_tail,
               it_tail):
    araw = pl.pallas_call(
        _score_body,
        out_shape=jax.ShapeDtypeStruct((AOUT,), jnp.float32),
        in_specs=[
            pl.BlockSpec(memory_space=pltpu.SMEM),
            pl.BlockSpec(memory_space=pltpu.SMEM),
            pl.BlockSpec(memory_space=pltpu.VMEM),
            pl.BlockSpec(memory_space=pl.ANY),
            pl.BlockSpec(memory_space=pl.ANY),
            pl.BlockSpec(memory_space=pltpu.VMEM),
            pl.BlockSpec(memory_space=pltpu.VMEM),
            pl.BlockSpec(memory_space=pltpu.VMEM),
            pl.BlockSpec(memory_space=pltpu.SMEM),
            pl.BlockSpec(memory_space=pltpu.VMEM),
            pl.BlockSpec(memory_space=pltpu.VMEM),
        ],
        out_specs=pl.BlockSpec(memory_space=pltpu.VMEM),
        scratch_shapes=[
            pltpu.VMEM((DIM, PAD, 128), jnp.float32),
            pltpu.VMEM((DIM, 128), jnp.float32),
            pltpu.SemaphoreType.DMA,
            pltpu.SemaphoreType.DMA,
        ],
    )(uidx16, idx_all, idx2d, utT, itT, W1, W2, b1, b2, ut_tail, it_tail)

    run = pl.kernel(
        _softmax_body,
        mesh=plsc.VectorSubcoreMesh(core_axis_name="c", subcore_axis_name="s"),
        out_type=jax.ShapeDtypeStruct((HIST,), jnp.float32),
        compiler_params=pltpu.CompilerParams(use_tc_tiling_on_sc=True),
        scratch_types=[
            pltpu.VMEM((AOUT,), jnp.float32),
        ],
    )
    return run(araw)


def kernel(user_indice, interacted_item_indices, user_table, item_table,
           W1, b1, W2, b2):
    idx_all = jnp.concatenate(
        [interacted_item_indices.astype(jnp.int32),
         jnp.zeros((PAD - HIST,), jnp.int32)])
    idx2d = idx_all.reshape(PAD, 1)
    uidx16 = jnp.full((16,), user_indice, dtype=jnp.int32)
    utT = user_table.T
    itT = item_table.T
    ntail = NROWS - TAILSTART
    ut_tail = jnp.pad(utT[:, TAILSTART:], ((0, 0), (0, 128 - ntail)))
    it_tail = jnp.pad(itT[:, TAILSTART:], ((0, 0), (0, 128 - ntail)))
    return _attention(uidx16, idx_all, idx2d, utT, itT, W1, W2, b1, b2,
                      ut_tail, it_tail)
